# baseline (speedup 1.0000x reference)
import functools

import jax
import jax.numpy as jnp
from jax import lax
from jax.experimental import pallas as pl
from jax.experimental.pallas import tpu as pltpu
from jax.experimental.pallas import tpu_sc as plsc

VOCAB = 1000000
EMB = 32
B = 16384
L = 50
N = B * L

NC, NS = 2, 16
NW = NC * NS
PER_W = N // NW
CHUNK = 1600
NCHUNK = PER_W // CHUNK


def _body(idx_hbm, tab_hbm, out_hbm, idx_v, rows_v, gsems, ssems):
    wid = lax.axis_index("s") * NC + lax.axis_index("c")
    base = wid * PER_W
    pltpu.sync_copy(idx_hbm.at[pl.ds(base, PER_W)], idx_v)

    def start_gather(g):
        return pltpu.async_copy(
            tab_hbm.at[idx_v.at[pl.ds(g * CHUNK, CHUNK)]],
            rows_v.at[g % 2],
            gsems.at[g % 2],
        )

    pending_store = [None, None]
    gather = [None, None]
    gather[0] = start_gather(0)
    for g in range(NCHUNK):
        s = g % 2
        gather[s].wait()
        if g + 1 < NCHUNK:
            if pending_store[1 - s] is not None:
                pending_store[1 - s].wait()
            gather[1 - s] = start_gather(g + 1)
        pending_store[s] = pltpu.async_copy(
            rows_v.at[s],
            out_hbm.at[pl.ds(base + g * CHUNK, CHUNK)],
            ssems.at[s],
        )
    for p in pending_store:
        if p is not None:
            p.wait()


@jax.jit
def _lookup(idx_flat, table):
    k = pl.kernel(
        _body,
        out_type=jax.ShapeDtypeStruct((N, EMB), jnp.float32),
        mesh=plsc.VectorSubcoreMesh(core_axis_name="c", subcore_axis_name="s"),
        compiler_params=pltpu.CompilerParams(use_tc_tiling_on_sc=False),
        scratch_types=[
            pltpu.VMEM((PER_W,), jnp.int32),
            pltpu.VMEM((2, CHUNK, EMB), jnp.float32),
            pltpu.SemaphoreType.DMA((2,)),
            pltpu.SemaphoreType.DMA((2,)),
        ],
    )
    return k(idx_flat, table)


def kernel(x, table):
    idx_flat = x.reshape(N).astype(jnp.int32)
    return _lookup(idx_flat, table).reshape(B, L, EMB)


# speedup vs baseline: 1.0024x; 1.0024x over previous
import functools

import jax
import jax.numpy as jnp
from jax import lax
from jax.experimental import pallas as pl
from jax.experimental.pallas import tpu as pltpu
from jax.experimental.pallas import tpu_sc as plsc

VOCAB = 1000000
EMB = 32
B = 16384
L = 50
N = B * L

NC, NS = 2, 16
NW = NC * NS
PER_W = N // NW
CHUNK = 800
NCHUNK = PER_W // CHUNK
NBUF = 4
DEPTH = 3


def _body(idx_hbm, tab_hbm, out_hbm, idx_v, rows_v, gsems, ssems):
    wid = lax.axis_index("s") * NC + lax.axis_index("c")
    base = wid * PER_W
    pltpu.sync_copy(idx_hbm.at[pl.ds(base, PER_W)], idx_v)

    def start_gather(g):
        return pltpu.async_copy(
            tab_hbm.at[idx_v.at[pl.ds(g * CHUNK, CHUNK)]],
            rows_v.at[g % NBUF],
            gsems.at[g % NBUF],
        )

    store = [None] * NBUF
    gather = [None] * NBUF
    for g in range(min(DEPTH, NCHUNK)):
        gather[g % NBUF] = start_gather(g)
    for g in range(NCHUNK):
        s = g % NBUF
        gather[s].wait()
        store[s] = pltpu.async_copy(
            rows_v.at[s],
            out_hbm.at[pl.ds(base + g * CHUNK, CHUNK)],
            ssems.at[s],
        )
        nxt = g + DEPTH
        if nxt < NCHUNK:
            ns = nxt % NBUF
            if store[ns] is not None:
                store[ns].wait()
            gather[ns] = start_gather(nxt)
    for p in store:
        if p is not None:
            p.wait()


@jax.jit
def _lookup(idx_flat, table):
    k = pl.kernel(
        _body,
        out_type=jax.ShapeDtypeStruct((N, EMB), jnp.float32),
        mesh=plsc.VectorSubcoreMesh(core_axis_name="c", subcore_axis_name="s"),
        compiler_params=pltpu.CompilerParams(use_tc_tiling_on_sc=False),
        scratch_types=[
            pltpu.VMEM((PER_W,), jnp.int32),
            pltpu.VMEM((NBUF, CHUNK, EMB), jnp.float32),
            pltpu.SemaphoreType.DMA((NBUF,)),
            pltpu.SemaphoreType.DMA((NBUF,)),
        ],
    )
    return k(idx_flat, table)


def kernel(x, table):
    idx_flat = x.reshape(N).astype(jnp.int32)
    return _lookup(idx_flat, table).reshape(B, L, EMB)


# speedup vs baseline: 1.7482x; 1.7441x over previous
import functools

import jax
import jax.numpy as jnp
from jax import lax
from jax.experimental import pallas as pl
from jax.experimental.pallas import tpu as pltpu
from jax.experimental.pallas import tpu_sc as plsc

VOCAB = 1000000
EMB = 32
B = 16384
L = 50
N = B * L

NC, NS = 2, 16
NW = NC * NS
PER_W = N // NW
CHUNK = 400
NCHUNK = PER_W // CHUNK
NBUF = 8
DEPTH = 6


def _body(idx_hbm, tab_hbm, out_hbm, idx_v, rows_v, gsems, ssems):
    wid = lax.axis_index("s") * NC + lax.axis_index("c")
    base = wid * PER_W
    pltpu.sync_copy(idx_hbm.at[pl.ds(base, PER_W)], idx_v)

    def start_gather(g):
        return pltpu.async_copy(
            tab_hbm.at[idx_v.at[pl.ds(g * CHUNK, CHUNK)]],
            rows_v.at[g % NBUF],
            gsems.at[g % NBUF],
        )

    store = [None] * NBUF
    gather = [None] * NBUF
    for g in range(min(DEPTH, NCHUNK)):
        gather[g % NBUF] = start_gather(g)
    for g in range(NCHUNK):
        s = g % NBUF
        gather[s].wait()
        store[s] = pltpu.async_copy(
            rows_v.at[s],
            out_hbm.at[pl.ds(base + g * CHUNK, CHUNK)],
            ssems.at[s],
        )
        nxt = g + DEPTH
        if nxt < NCHUNK:
            ns = nxt % NBUF
            if store[ns] is not None:
                store[ns].wait()
            gather[ns] = start_gather(nxt)
    for p in store:
        if p is not None:
            p.wait()


@jax.jit
def _lookup(idx_flat, table):
    k = pl.kernel(
        _body,
        out_type=jax.ShapeDtypeStruct((N, EMB), jnp.float32),
        mesh=plsc.VectorSubcoreMesh(core_axis_name="c", subcore_axis_name="s"),
        compiler_params=pltpu.CompilerParams(use_tc_tiling_on_sc=False),
        scratch_types=[
            pltpu.VMEM((PER_W,), jnp.int32),
            pltpu.VMEM((NBUF, CHUNK, EMB), jnp.float32),
            pltpu.SemaphoreType.DMA((NBUF,)),
            pltpu.SemaphoreType.DMA((NBUF,)),
        ],
    )
    return k(idx_flat, table)


def kernel(x, table):
    idx_flat = x.T.reshape(N).astype(jnp.int32)
    out = _lookup(idx_flat, table)
    return out.reshape(L, B, EMB).transpose(1, 0, 2)
